# Initial kernel scaffold; baseline (speedup 1.0000x reference)
import functools

import jax
import jax.numpy as jnp
from jax import lax
from jax.experimental import pallas as pl
from jax.experimental.pallas import tpu as pltpu
from jax.experimental.pallas import tpu_sc as plsc

N = 10000
D = 128
E = 320000
EPS = 1e-5

NC = 2
NS = 16
NW = NC * NS
CH = 128
K_AGG = 79
K_DEG = 158
E_PAD = NW * K_AGG * CH
ROWS_SH = 10016
RPT = ROWS_SH // NS


def _sc_degrees(deg_idx, ones_hbm, zeros_hbm):
  mesh = plsc.VectorSubcoreMesh(core_axis_name="c", subcore_axis_name="s")

  @functools.partial(
      pl.kernel,
      out_type=jax.ShapeDtypeStruct((NC, ROWS_SH, 16), jnp.float32),
      mesh=mesh,
      scratch_types=[
          pltpu.VMEM((K_DEG, CH), jnp.int32),
          pltpu.VMEM((CH, 16), jnp.float32),
          pltpu.VMEM_SHARED((ROWS_SH, 16), jnp.float32),
      ],
  )
  def k(idx_hbm, ones_h, zeros_h, out_hbm, idx_v, ones_v, deg_sh):
    c = lax.axis_index("c")
    s = lax.axis_index("s")
    pltpu.sync_copy(idx_hbm.at[c, s], idx_v)
    pltpu.sync_copy(ones_h, ones_v)
    pltpu.sync_copy(zeros_h, deg_sh.at[pl.ds(s * RPT, RPT)])
    plsc.subcore_barrier()

    @pl.loop(0, K_DEG)
    def _(j):
      pltpu.sync_copy(ones_v, deg_sh.at[idx_v.at[j]], add=True)

    plsc.subcore_barrier()
    pltpu.sync_copy(deg_sh.at[pl.ds(s * RPT, RPT)],
                    out_hbm.at[c, pl.ds(s * RPT, RPT)])

  return k(deg_idx, ones_hbm, zeros_hbm)


def _sc_aggregate(h_pad, src_t, dst_t, zeros_hbm):
  mesh = plsc.VectorSubcoreMesh(core_axis_name="c", subcore_axis_name="s")

  @functools.partial(
      pl.kernel,
      out_type=jax.ShapeDtypeStruct((NC, ROWS_SH, D), jnp.float32),
      mesh=mesh,
      scratch_types=[
          pltpu.VMEM((K_AGG, CH), jnp.int32),
          pltpu.VMEM((K_AGG, CH), jnp.int32),
          pltpu.VMEM((CH, D), jnp.float32),
          pltpu.VMEM_SHARED((ROWS_SH, D), jnp.float32),
          pltpu.SemaphoreType.DMA,
      ],
  )
  def k(h_hbm, src_h, dst_h, z_h, out_hbm, sidx_v, didx_v, rows_v, agg_sh,
        sem):
    c = lax.axis_index("c")
    s = lax.axis_index("s")
    wid = c * NS + s
    pltpu.sync_copy(src_h.at[wid], sidx_v)
    pltpu.sync_copy(dst_h.at[wid], didx_v)
    pltpu.sync_copy(z_h, agg_sh.at[pl.ds(s * RPT, RPT)])
    plsc.subcore_barrier()

    @pl.loop(0, K_AGG)
    def _(j):
      pltpu.async_copy(h_hbm.at[sidx_v.at[j]], rows_v, sem).wait()
      pltpu.sync_copy(rows_v, agg_sh.at[didx_v.at[j]], add=True)

    plsc.subcore_barrier()
    pltpu.sync_copy(agg_sh.at[pl.ds(s * RPT, RPT)],
                    out_hbm.at[c, pl.ds(s * RPT, RPT)])

  return k(h_pad, src_t, dst_t, zeros_hbm)


def _tc_pre(x_pad, w1, deg):
  def body(x_ref, w_ref, d_ref, o_ref):
    h = jnp.dot(x_ref[...], w_ref[...], preferred_element_type=jnp.float32)
    cnt = d_ref[0, :, 0:1]
    o_ref[...] = h * lax.rsqrt(jnp.maximum(cnt, 1.0))

  return pl.pallas_call(
      body,
      out_shape=jax.ShapeDtypeStruct((ROWS_SH, D), jnp.float32),
  )(x_pad, w1, deg)


def _tc_mid(p, deg, b1, g1, be1, w2):
  def body(p_ref, d_ref, b_ref, g_ref, be_ref, w_ref, o_ref):
    agg = p_ref[0, :N, :] + p_ref[1, :N, :]
    in_cnt = d_ref[1, :N, 0:1]
    agg = agg * lax.rsqrt(jnp.maximum(in_cnt, 1.0)) + b_ref[...]
    a = jnp.where(agg > 0, agg, jnp.expm1(agg))
    mean = jnp.mean(a, axis=0, keepdims=True)
    var = jnp.mean((a - mean) ** 2, axis=0, keepdims=True)
    h1 = (a - mean) * lax.rsqrt(var + EPS) * g_ref[...] + be_ref[...]
    out_cnt = d_ref[0, :N, 0:1]
    h2 = jnp.dot(h1, w_ref[...], preferred_element_type=jnp.float32)
    h2 = h2 * lax.rsqrt(jnp.maximum(out_cnt, 1.0))
    o_ref[0:N, :] = h2
    o_ref[N:ROWS_SH, :] = jnp.zeros((ROWS_SH - N, D), jnp.float32)

  return pl.pallas_call(
      body,
      out_shape=jax.ShapeDtypeStruct((ROWS_SH, D), jnp.float32),
  )(p, deg, b1, g1, be1, w2)


def _tc_post(p, deg, b2, g2, be2):
  def body(p_ref, d_ref, b_ref, g_ref, be_ref, o_ref):
    agg = p_ref[0, :N, :] + p_ref[1, :N, :]
    in_cnt = d_ref[1, :N, 0:1]
    agg = agg * lax.rsqrt(jnp.maximum(in_cnt, 1.0)) + b_ref[...]
    a = jnp.where(agg > 0, agg, jnp.expm1(agg))
    mean = jnp.mean(a, axis=0, keepdims=True)
    var = jnp.mean((a - mean) ** 2, axis=0, keepdims=True)
    o_ref[...] = (a - mean) * lax.rsqrt(var + EPS) * g_ref[...] + be_ref[...]

  return pl.pallas_call(
      body,
      out_shape=jax.ShapeDtypeStruct((N, D), jnp.float32),
  )(p, deg, b2, g2, be2)


@jax.jit
def kernel(features, edge_index, W1, b1, gamma1, beta1, W2, b2, gamma2,
           beta2):
  src = edge_index[0].astype(jnp.int32)
  dst = edge_index[1].astype(jnp.int32)
  pad = jnp.full((E_PAD - E,), N, jnp.int32)
  src_p = jnp.concatenate([src, pad])
  dst_p = jnp.concatenate([dst, pad])
  src_t = src_p.reshape(NW, K_AGG, CH)
  dst_t = dst_p.reshape(NW, K_AGG, CH)
  deg_idx = jnp.stack([src_p, dst_p]).reshape(2, NS, K_DEG, CH)

  ones16 = jnp.ones((CH, 16), jnp.float32)
  zeros16 = jnp.zeros((RPT, 16), jnp.float32)
  zerosd = jnp.zeros((RPT, D), jnp.float32)
  x_pad = jnp.concatenate(
      [features, jnp.zeros((ROWS_SH - N, D), jnp.float32)])

  deg = _sc_degrees(deg_idx, ones16, zeros16)
  h1s = _tc_pre(x_pad, W1, deg)
  p1 = _sc_aggregate(h1s, src_t, dst_t, zerosd)
  h2s = _tc_mid(p1, deg, b1.reshape(1, D), gamma1.reshape(1, D),
                beta1.reshape(1, D), W2)
  p2 = _sc_aggregate(h2s, src_t, dst_t, zerosd)
  return _tc_post(p2, deg, b2.reshape(1, D), gamma2.reshape(1, D),
                  beta2.reshape(1, D))


# speedup vs baseline: 3.6965x; 3.6965x over previous
import functools

import jax
import jax.numpy as jnp
from jax import lax
from jax.experimental import pallas as pl
from jax.experimental.pallas import tpu as pltpu
from jax.experimental.pallas import tpu_sc as plsc

N = 10000
D = 128
E = 320000
EPS = 1e-5

NC = 2
NS = 16
NW = NC * NS
CH = 128
K_AGG = 79
K_DEG = 158
E_PAD = NW * K_AGG * CH
ROWS_SH = 10112
RPT = ROWS_SH // NS


def _sc_degrees(deg_idx, ones_hbm, zeros_hbm):
  mesh = plsc.VectorSubcoreMesh(core_axis_name="c", subcore_axis_name="s")

  @functools.partial(
      pl.kernel,
      out_type=jax.ShapeDtypeStruct((NC, ROWS_SH, D), jnp.float32),
      mesh=mesh,
      scratch_types=[
          pltpu.VMEM((K_DEG, CH), jnp.int32),
          pltpu.VMEM((CH, D), jnp.float32),
          pltpu.VMEM_SHARED((ROWS_SH, D), jnp.float32),
      ],
  )
  def k(idx_hbm, ones_h, zeros_h, out_hbm, idx_v, ones_v, deg_sh):
    c = lax.axis_index("c")
    s = lax.axis_index("s")
    pltpu.sync_copy(idx_hbm.at[c, s], idx_v)
    pltpu.sync_copy(ones_h, ones_v)
    pltpu.sync_copy(zeros_h, deg_sh.at[pl.ds(s * RPT, RPT)])
    plsc.subcore_barrier()

    @pl.loop(0, K_DEG)
    def _(j):
      pltpu.sync_copy(ones_v, deg_sh.at[idx_v.at[j]], add=True)

    plsc.subcore_barrier()
    pltpu.sync_copy(deg_sh.at[pl.ds(s * RPT, RPT)],
                    out_hbm.at[c, pl.ds(s * RPT, RPT)])

  return k(deg_idx, ones_hbm, zeros_hbm)


def _sc_aggregate(h_pad, src_t, dst_t, zeros_hbm):
  mesh = plsc.VectorSubcoreMesh(core_axis_name="c", subcore_axis_name="s")

  @functools.partial(
      pl.kernel,
      out_type=jax.ShapeDtypeStruct((NC, ROWS_SH, D), jnp.float32),
      mesh=mesh,
      scratch_types=[
          pltpu.VMEM((K_AGG, CH), jnp.int32),
          pltpu.VMEM((K_AGG, CH), jnp.int32),
          pltpu.VMEM((CH, D), jnp.float32),
          pltpu.VMEM_SHARED((ROWS_SH, D), jnp.float32),
          pltpu.SemaphoreType.DMA,
      ],
  )
  def k(h_hbm, src_h, dst_h, z_h, out_hbm, sidx_v, didx_v, rows_v, agg_sh,
        sem):
    c = lax.axis_index("c")
    s = lax.axis_index("s")
    wid = c * NS + s
    pltpu.sync_copy(src_h.at[wid], sidx_v)
    pltpu.sync_copy(dst_h.at[wid], didx_v)
    pltpu.sync_copy(z_h, agg_sh.at[pl.ds(s * RPT, RPT)])
    plsc.subcore_barrier()

    @pl.loop(0, K_AGG)
    def _(j):
      pltpu.async_copy(h_hbm.at[sidx_v.at[j]], rows_v, sem).wait()
      pltpu.sync_copy(rows_v, agg_sh.at[didx_v.at[j]], add=True)

    plsc.subcore_barrier()
    pltpu.sync_copy(agg_sh.at[pl.ds(s * RPT, RPT)],
                    out_hbm.at[c, pl.ds(s * RPT, RPT)])

  return k(h_pad, src_t, dst_t, zeros_hbm)


def _tc_pre(x_pad, w1, deg):
  def body(x_ref, w_ref, d_ref, o_ref):
    h = jnp.dot(x_ref[...], w_ref[...], preferred_element_type=jnp.float32)
    cnt = d_ref[0, :, 0:1]
    o_ref[...] = h * lax.rsqrt(jnp.maximum(cnt, 1.0))

  return pl.pallas_call(
      body,
      out_shape=jax.ShapeDtypeStruct((ROWS_SH, D), jnp.float32),
  )(x_pad, w1, deg)


def _tc_mid(p, deg, b1, g1, be1, w2):
  def body(p_ref, d_ref, b_ref, g_ref, be_ref, w_ref, o_ref):
    agg = p_ref[0, :N, :] + p_ref[1, :N, :]
    in_cnt = d_ref[1, :N, 0:1]
    agg = agg * lax.rsqrt(jnp.maximum(in_cnt, 1.0)) + b_ref[...]
    a = jnp.where(agg > 0, agg, jnp.exp(agg) - 1.0)
    mean = jnp.mean(a, axis=0, keepdims=True)
    var = jnp.mean((a - mean) ** 2, axis=0, keepdims=True)
    h1 = (a - mean) * lax.rsqrt(var + EPS) * g_ref[...] + be_ref[...]
    out_cnt = d_ref[0, :N, 0:1]
    h2 = jnp.dot(h1, w_ref[...], preferred_element_type=jnp.float32)
    h2 = h2 * lax.rsqrt(jnp.maximum(out_cnt, 1.0))
    o_ref[0:N, :] = h2
    o_ref[N:ROWS_SH, :] = jnp.zeros((ROWS_SH - N, D), jnp.float32)

  return pl.pallas_call(
      body,
      out_shape=jax.ShapeDtypeStruct((ROWS_SH, D), jnp.float32),
  )(p, deg, b1, g1, be1, w2)


def _tc_post(p, deg, b2, g2, be2):
  def body(p_ref, d_ref, b_ref, g_ref, be_ref, o_ref):
    agg = p_ref[0, :N, :] + p_ref[1, :N, :]
    in_cnt = d_ref[1, :N, 0:1]
    agg = agg * lax.rsqrt(jnp.maximum(in_cnt, 1.0)) + b_ref[...]
    a = jnp.where(agg > 0, agg, jnp.exp(agg) - 1.0)
    mean = jnp.mean(a, axis=0, keepdims=True)
    var = jnp.mean((a - mean) ** 2, axis=0, keepdims=True)
    o_ref[...] = (a - mean) * lax.rsqrt(var + EPS) * g_ref[...] + be_ref[...]

  return pl.pallas_call(
      body,
      out_shape=jax.ShapeDtypeStruct((N, D), jnp.float32),
  )(p, deg, b2, g2, be2)


@jax.jit
def kernel(features, edge_index, W1, b1, gamma1, beta1, W2, b2, gamma2,
           beta2):
  src = edge_index[0].astype(jnp.int32)
  dst = edge_index[1].astype(jnp.int32)
  pad = jnp.full((E_PAD - E,), N, jnp.int32)
  src_p = jnp.concatenate([src, pad])
  dst_p = jnp.concatenate([dst, pad])
  src_t = src_p.reshape(NW, K_AGG, CH)
  dst_t = dst_p.reshape(NW, K_AGG, CH)
  deg_idx = jnp.stack([src_p, dst_p]).reshape(2, NS, K_DEG, CH)

  onesd = jnp.ones((CH, D), jnp.float32)
  zerosd = jnp.zeros((RPT, D), jnp.float32)
  x_pad = jnp.concatenate(
      [features, jnp.zeros((ROWS_SH - N, D), jnp.float32)])

  deg = _sc_degrees(deg_idx, onesd, zerosd)
  h1s = _tc_pre(x_pad, W1, deg)
  p1 = _sc_aggregate(h1s, src_t, dst_t, zerosd)
  h2s = _tc_mid(p1, deg, b1.reshape(1, D), gamma1.reshape(1, D),
                beta1.reshape(1, D), W2)
  p2 = _sc_aggregate(h2s, src_t, dst_t, zerosd)
  return _tc_post(p2, deg, b2.reshape(1, D), gamma2.reshape(1, D),
                  beta2.reshape(1, D))
